# baseline (speedup 1.0000x reference)
import functools

import jax
import jax.numpy as jnp
from jax import lax
from jax.experimental import pallas as pl
from jax.experimental.pallas import tpu as pltpu
from jax.experimental.pallas import tpu_sc as plsc

N_NODES = 10000
N_EDGES = 320000
N_GRAPHS = 64
D_HID = 32

NC = 2
NS = 16
NW = NC * NS
ROW = 128
RPW = 80
ROWS = NW * RPW
EP = ROWS * ROW
NPAD = 10240
RZ = NPAD // NS
GK = 8
NGRP = RPW // GK

_mesh = plsc.VectorSubcoreMesh(core_axis_name="c", subcore_axis_name="s")
_sc_params = pltpu.CompilerParams(use_tc_tiling_on_sc=False)


@functools.partial(
    pl.kernel,
    out_type=jax.ShapeDtypeStruct((NC, NPAD), jnp.float32),
    mesh=_mesh,
    compiler_params=_sc_params,
    scratch_types=[
        pltpu.VMEM((RPW, ROW), jnp.int32),
        pltpu.VMEM((ROW,), jnp.float32),
        pltpu.VMEM((RZ,), jnp.float32),
        pltpu.VMEM_SHARED((NPAD,), jnp.float32),
        pltpu.SemaphoreType.DMA,
    ],
)
def _sc_count(dst_hbm, out_hbm, dst_v, ones_v, zer_v, cnt_sh, csem):
    c = lax.axis_index("c")
    s = lax.axis_index("s")
    wid = s * NC + c
    zero16 = jnp.zeros((16,), jnp.float32)
    ones16 = jnp.ones((16,), jnp.float32)

    ld_dst = pltpu.async_copy(dst_hbm.at[pl.ds(wid * RPW, RPW)], dst_v, csem)

    def fill(i, _):
        zer_v[pl.ds(i * 16, 16)] = zero16
        return 0

    lax.fori_loop(0, RZ // 16, fill, 0)
    for k in range(ROW // 16):
        ones_v[pl.ds(k * 16, 16)] = ones16
    pltpu.sync_copy(zer_v, cnt_sh.at[pl.ds(s * RZ, RZ)])
    ld_dst.wait()
    plsc.subcore_barrier()

    def body(g, _):
        descs = [
            pltpu.async_copy(ones_v, cnt_sh.at[dst_v.at[g * 16 + t]],
                             csem, add=True)
            for t in range(16)
        ]
        for d in descs:
            d.wait()
        return 0

    lax.fori_loop(0, RPW // 16, body, 0)
    plsc.subcore_barrier()
    pltpu.sync_copy(cnt_sh.at[pl.ds(s * RZ, RZ)],
                    out_hbm.at[c].at[pl.ds(s * RZ, RZ)])


@functools.partial(
    pl.kernel,
    out_type=jax.ShapeDtypeStruct((NC, NPAD, D_HID), jnp.float32),
    mesh=_mesh,
    compiler_params=_sc_params,
    scratch_types=[
        pltpu.VMEM((RPW, ROW), jnp.int32),
        pltpu.VMEM((RPW, ROW), jnp.int32),
        pltpu.VMEM((2, GK, ROW, D_HID), jnp.float32),
        pltpu.VMEM_SHARED((NPAD, D_HID), jnp.float32),
        pltpu.VMEM_SHARED((NPAD, D_HID), jnp.float32),
        pltpu.SemaphoreType.DMA,
        pltpu.SemaphoreType.DMA,
        pltpu.SemaphoreType.DMA,
        pltpu.SemaphoreType.DMA,
    ],
)
def _sc_msgpass(hp_hbm, src_hbm, dst_hbm, out_hbm,
                src_v, dst_v, bufs, agg_sh, hp_sh,
                gsem0, gsem1, ssem0, ssem1):
    c = lax.axis_index("c")
    s = lax.axis_index("s")
    wid = s * NC + c
    zero16 = jnp.zeros((16,), jnp.float32)
    zbuf = bufs.at[0].at[0]

    def zbody(j, _):
        r = zbuf.at[j]
        r[pl.ds(0, 16)] = zero16
        r[pl.ds(16, 16)] = zero16
        return 0

    stage = pltpu.async_copy(hp_hbm.at[pl.ds(s * RZ, RZ)],
                             hp_sh.at[pl.ds(s * RZ, RZ)], gsem0)
    ld_src = pltpu.async_copy(src_hbm.at[pl.ds(wid * RPW, RPW)], src_v, ssem0)
    ld_dst = pltpu.async_copy(dst_hbm.at[pl.ds(wid * RPW, RPW)], dst_v, ssem1)
    lax.fori_loop(0, ROW, zbody, 0)
    for t in range(RZ // ROW):
        pltpu.sync_copy(zbuf, agg_sh.at[pl.ds(s * RZ + t * ROW, ROW)])
    stage.wait()
    ld_src.wait()
    ld_dst.wait()
    plsc.subcore_barrier()

    gsems = (gsem0, gsem1)

    def gather_group(g, bank):
        for t in range(GK):
            pltpu.async_copy(hp_sh.at[src_v.at[g * GK + t]],
                             bufs.at[bank].at[t], gsems[bank])

    def wait_gathers(g, bank):
        for t in range(GK):
            pltpu.make_async_copy(hp_sh.at[src_v.at[g * GK + t]],
                                  bufs.at[bank].at[t], gsems[bank]).wait()

    def fire_scatters(g, bank, ssem):
        return [
            pltpu.async_copy(bufs.at[bank].at[t],
                             agg_sh.at[dst_v.at[g * GK + t]], ssem, add=True)
            for t in range(GK)
        ]

    gather_group(0, 0)

    def body(i, _):
        g = 2 * i
        gather_group(g + 1, 1)
        wait_gathers(g, 0)
        sc_a = fire_scatters(g, 0, ssem0)
        wait_gathers(g + 1, 1)
        sc_b = fire_scatters(g + 1, 1, ssem1)
        for d in sc_a:
            d.wait()
        gather_group(g + 2, 0)
        for d in sc_b:
            d.wait()
        return 0

    lax.fori_loop(0, NGRP // 2 - 1, body, 0)
    gather_group(NGRP - 1, 1)
    wait_gathers(NGRP - 2, 0)
    sc_a = fire_scatters(NGRP - 2, 0, ssem0)
    wait_gathers(NGRP - 1, 1)
    sc_b = fire_scatters(NGRP - 1, 1, ssem1)
    for d in sc_a + sc_b:
        d.wait()

    plsc.subcore_barrier()
    pltpu.sync_copy(agg_sh.at[pl.ds(s * RZ, RZ)],
                    out_hbm.at[c].at[pl.ds(s * RZ, RZ)])


def _tc_prescale_body(x_ref, w1_ref, cnt_ref, hp_ref, dinv_ref):
    xw = jnp.dot(x_ref[...], w1_ref[...], preferred_element_type=jnp.float32)
    cnt = cnt_ref[0, :] + cnt_ref[1, :]
    dinv = lax.rsqrt(cnt[:N_NODES] + 1.0)[:, None]
    hp_ref[:N_NODES, :] = xw * dinv
    dinv_ref[...] = dinv


def _tc_layer2_body(agg_ref, hp1_ref, dinv_ref, b1_ref, w2_ref, hp2_ref):
    agg = agg_ref[0, :N_NODES, :] + agg_ref[1, :N_NODES, :]
    conv1 = dinv_ref[...] * (agg + hp1_ref[:N_NODES, :]) + b1_ref[...]
    h1 = jnp.maximum(conv1, 0.0)
    xw2 = jnp.dot(h1, w2_ref[...], preferred_element_type=jnp.float32)
    hp2_ref[:N_NODES, :] = xw2 * dinv_ref[...]


def _tc_head_body(agg_ref, hp2_ref, dinv_ref, b2_ref, batch_ref,
                  wc_ref, bc_ref, out_ref):
    agg = agg_ref[0, :N_NODES, :] + agg_ref[1, :N_NODES, :]
    conv2 = dinv_ref[...] * (agg + hp2_ref[:N_NODES, :]) + b2_ref[...]
    gids = lax.broadcasted_iota(jnp.int32, (N_GRAPHS, N_NODES), 0)
    mask = (batch_ref[...] == gids).astype(jnp.float32)
    sums = jnp.dot(mask, conv2, preferred_element_type=jnp.float32)
    cnt = jnp.sum(mask, axis=1, keepdims=True)
    pooled = sums / jnp.maximum(cnt, 1.0)
    out_ref[...] = (
        jnp.dot(pooled, wc_ref[...], preferred_element_type=jnp.float32)
        + bc_ref[...])


@jax.jit
def kernel(x, edge_index, batch, W1, b1, W2, b2, Wc, bc):
    src = edge_index[0].astype(jnp.int32)
    dst = edge_index[1].astype(jnp.int32)
    pad = EP - N_EDGES
    src2d = jnp.concatenate(
        [src, jnp.zeros((pad,), jnp.int32)]).reshape(ROWS, ROW)
    dst2d = jnp.concatenate(
        [dst, jnp.full((pad,), N_NODES, jnp.int32)]).reshape(ROWS, ROW)

    cnt_parts = _sc_count(dst2d)

    hp1, dinv = pl.pallas_call(
        _tc_prescale_body,
        out_shape=(
            jax.ShapeDtypeStruct((NPAD, D_HID), jnp.float32),
            jax.ShapeDtypeStruct((N_NODES, 1), jnp.float32),
        ),
    )(x, W1, cnt_parts)

    agg1 = _sc_msgpass(hp1, src2d, dst2d)

    hp2 = pl.pallas_call(
        _tc_layer2_body,
        out_shape=jax.ShapeDtypeStruct((NPAD, D_HID), jnp.float32),
    )(agg1, hp1, dinv, b1.reshape(1, D_HID), W2)

    agg2 = _sc_msgpass(hp2, src2d, dst2d)

    out = pl.pallas_call(
        _tc_head_body,
        out_shape=jax.ShapeDtypeStruct((N_GRAPHS, bc.shape[0]), jnp.float32),
    )(agg2, hp2, dinv, b2.reshape(1, D_HID), batch.astype(jnp.int32)[None, :],
      Wc, bc.reshape(1, bc.shape[0]))
    return out


# speedup vs baseline: 1.1261x; 1.1261x over previous
import functools

import jax
import jax.numpy as jnp
from jax import lax
from jax.experimental import pallas as pl
from jax.experimental.pallas import tpu as pltpu
from jax.experimental.pallas import tpu_sc as plsc

N_NODES = 10000
N_EDGES = 320000
N_GRAPHS = 64
D_HID = 32

NC = 2
NS = 16
NW = NC * NS
ROW = 128
RPW = 80
ROWS = NW * RPW
EP = ROWS * ROW
NPAD = 10240
RZ = NPAD // NS
GK = 8
NGRP = RPW // GK

_mesh = plsc.VectorSubcoreMesh(core_axis_name="c", subcore_axis_name="s")
_sc_params = pltpu.CompilerParams(use_tc_tiling_on_sc=False)


@functools.partial(
    pl.kernel,
    out_type=jax.ShapeDtypeStruct((NC, NPAD), jnp.float32),
    mesh=_mesh,
    compiler_params=_sc_params,
    scratch_types=[
        pltpu.VMEM((RPW, ROW), jnp.int32),
        pltpu.VMEM((ROW,), jnp.float32),
        pltpu.VMEM((RZ,), jnp.float32),
        pltpu.VMEM_SHARED((NPAD,), jnp.float32),
        pltpu.SemaphoreType.DMA,
    ],
)
def _sc_count(dst_hbm, out_hbm, dst_v, ones_v, zer_v, cnt_sh, csem):
    c = lax.axis_index("c")
    s = lax.axis_index("s")
    wid = s * NC + c
    zero16 = jnp.zeros((16,), jnp.float32)
    ones16 = jnp.ones((16,), jnp.float32)

    ld_dst = pltpu.async_copy(dst_hbm.at[pl.ds(wid * RPW, RPW)], dst_v, csem)

    def fill(i, _):
        zer_v[pl.ds(i * 16, 16)] = zero16
        return 0

    lax.fori_loop(0, RZ // 16, fill, 0)
    for k in range(ROW // 16):
        ones_v[pl.ds(k * 16, 16)] = ones16
    pltpu.sync_copy(zer_v, cnt_sh.at[pl.ds(s * RZ, RZ)])
    ld_dst.wait()
    plsc.subcore_barrier()

    def body(g, _):
        descs = [
            pltpu.async_copy(ones_v, cnt_sh.at[dst_v.at[g * 16 + t]],
                             csem, add=True)
            for t in range(16)
        ]
        for d in descs:
            d.wait()
        return 0

    lax.fori_loop(0, RPW // 16, body, 0)
    plsc.subcore_barrier()
    pltpu.sync_copy(cnt_sh.at[pl.ds(s * RZ, RZ)],
                    out_hbm.at[c].at[pl.ds(s * RZ, RZ)])


@functools.partial(
    pl.kernel,
    out_type=jax.ShapeDtypeStruct((NC, NPAD, D_HID), jnp.float32),
    mesh=_mesh,
    compiler_params=_sc_params,
    scratch_types=[
        pltpu.VMEM((RPW, ROW), jnp.int32),
        pltpu.VMEM((RPW, ROW), jnp.int32),
        pltpu.VMEM((2, GK, ROW, D_HID), jnp.float32),
        pltpu.VMEM_SHARED((NPAD, D_HID), jnp.float32),
        pltpu.VMEM_SHARED((NPAD, D_HID), jnp.float32),
        pltpu.SemaphoreType.DMA,
        pltpu.SemaphoreType.DMA,
        pltpu.SemaphoreType.DMA,
        pltpu.SemaphoreType.DMA,
    ],
)
def _sc_msgpass(hp_hbm, src_hbm, dst_hbm, out_hbm,
                src_v, dst_v, bufs, agg_sh, hp_sh,
                gsem0, gsem1, ssem0, ssem1):
    c = lax.axis_index("c")
    s = lax.axis_index("s")
    wid = s * NC + c
    zero16 = jnp.zeros((16,), jnp.float32)
    zbuf = bufs.at[0].at[0]

    def zbody(j, _):
        r = zbuf.at[j]
        r[pl.ds(0, 16)] = zero16
        r[pl.ds(16, 16)] = zero16
        return 0

    stage = pltpu.async_copy(hp_hbm.at[pl.ds(s * RZ, RZ)],
                             hp_sh.at[pl.ds(s * RZ, RZ)], gsem0)
    ld_src = pltpu.async_copy(src_hbm.at[pl.ds(wid * RPW, RPW)], src_v, ssem0)
    ld_dst = pltpu.async_copy(dst_hbm.at[pl.ds(wid * RPW, RPW)], dst_v, ssem1)
    lax.fori_loop(0, ROW, zbody, 0)
    for t in range(RZ // ROW):
        pltpu.sync_copy(zbuf, agg_sh.at[pl.ds(s * RZ + t * ROW, ROW)])
    stage.wait()
    ld_src.wait()
    ld_dst.wait()
    plsc.subcore_barrier()

    gsems = (gsem0, gsem1)

    def gather_group(g, bank):
        for t in range(GK):
            pltpu.async_copy(hp_sh.at[src_v.at[g * GK + t]],
                             bufs.at[bank].at[t], gsems[bank])

    def wait_gathers(g, bank):
        for t in range(GK):
            pltpu.make_async_copy(hp_sh.at[src_v.at[g * GK + t]],
                                  bufs.at[bank].at[t], gsems[bank]).wait()

    def scatter_group(g, bank, ssem):
        descs = [
            pltpu.async_copy(bufs.at[bank].at[t],
                             agg_sh.at[dst_v.at[g * GK + t]], ssem, add=True)
            for t in range(GK)
        ]
        for d in descs:
            d.wait()

    gather_group(0, 0)

    def body(i, _):
        g = 2 * i
        gather_group(g + 1, 1)
        wait_gathers(g, 0)
        scatter_group(g, 0, ssem0)
        gather_group(g + 2, 0)
        wait_gathers(g + 1, 1)
        scatter_group(g + 1, 1, ssem1)
        return 0

    lax.fori_loop(0, NGRP // 2 - 1, body, 0)
    gather_group(NGRP - 1, 1)
    wait_gathers(NGRP - 2, 0)
    scatter_group(NGRP - 2, 0, ssem0)
    wait_gathers(NGRP - 1, 1)
    scatter_group(NGRP - 1, 1, ssem1)

    plsc.subcore_barrier()
    pltpu.sync_copy(agg_sh.at[pl.ds(s * RZ, RZ)],
                    out_hbm.at[c].at[pl.ds(s * RZ, RZ)])


def _tc_prescale_body(x_ref, w1_ref, cnt_ref, hp_ref, dinv_ref):
    xw = jnp.dot(x_ref[...], w1_ref[...], preferred_element_type=jnp.float32)
    cnt = cnt_ref[0, :] + cnt_ref[1, :]
    dinv = lax.rsqrt(cnt[:N_NODES] + 1.0)[:, None]
    hp_ref[:N_NODES, :] = xw * dinv
    dinv_ref[...] = dinv


def _tc_layer2_body(agg_ref, hp1_ref, dinv_ref, b1_ref, w2_ref, hp2_ref):
    agg = agg_ref[0, :N_NODES, :] + agg_ref[1, :N_NODES, :]
    conv1 = dinv_ref[...] * (agg + hp1_ref[:N_NODES, :]) + b1_ref[...]
    h1 = jnp.maximum(conv1, 0.0)
    xw2 = jnp.dot(h1, w2_ref[...], preferred_element_type=jnp.float32)
    hp2_ref[:N_NODES, :] = xw2 * dinv_ref[...]


def _tc_head_body(agg_ref, hp2_ref, dinv_ref, b2_ref, batch_ref,
                  wc_ref, bc_ref, out_ref):
    agg = agg_ref[0, :N_NODES, :] + agg_ref[1, :N_NODES, :]
    conv2 = dinv_ref[...] * (agg + hp2_ref[:N_NODES, :]) + b2_ref[...]
    gids = lax.broadcasted_iota(jnp.int32, (N_GRAPHS, N_NODES), 0)
    mask = (batch_ref[...] == gids).astype(jnp.float32)
    sums = jnp.dot(mask, conv2, preferred_element_type=jnp.float32)
    cnt = jnp.sum(mask, axis=1, keepdims=True)
    pooled = sums / jnp.maximum(cnt, 1.0)
    out_ref[...] = (
        jnp.dot(pooled, wc_ref[...], preferred_element_type=jnp.float32)
        + bc_ref[...])


@jax.jit
def kernel(x, edge_index, batch, W1, b1, W2, b2, Wc, bc):
    src = edge_index[0].astype(jnp.int32)
    dst = edge_index[1].astype(jnp.int32)
    pad = EP - N_EDGES
    src2d = jnp.concatenate(
        [src, jnp.zeros((pad,), jnp.int32)]).reshape(ROWS, ROW)
    dst2d = jnp.concatenate(
        [dst, jnp.full((pad,), N_NODES, jnp.int32)]).reshape(ROWS, ROW)

    cnt_parts = _sc_count(dst2d)

    hp1, dinv = pl.pallas_call(
        _tc_prescale_body,
        out_shape=(
            jax.ShapeDtypeStruct((NPAD, D_HID), jnp.float32),
            jax.ShapeDtypeStruct((N_NODES, 1), jnp.float32),
        ),
    )(x, W1, cnt_parts)

    agg1 = _sc_msgpass(hp1, src2d, dst2d)

    hp2 = pl.pallas_call(
        _tc_layer2_body,
        out_shape=jax.ShapeDtypeStruct((NPAD, D_HID), jnp.float32),
    )(agg1, hp1, dinv, b1.reshape(1, D_HID), W2)

    agg2 = _sc_msgpass(hp2, src2d, dst2d)

    out = pl.pallas_call(
        _tc_head_body,
        out_shape=jax.ShapeDtypeStruct((N_GRAPHS, bc.shape[0]), jnp.float32),
    )(agg2, hp2, dinv, b2.reshape(1, D_HID), batch.astype(jnp.int32)[None, :],
      Wc, bc.reshape(1, bc.shape[0]))
    return out


# speedup vs baseline: 1.1325x; 1.0057x over previous
import functools

import jax
import jax.numpy as jnp
from jax import lax
from jax.experimental import pallas as pl
from jax.experimental.pallas import tpu as pltpu
from jax.experimental.pallas import tpu_sc as plsc

N_NODES = 10000
N_EDGES = 320000
N_GRAPHS = 64
D_HID = 32

NC = 2
NS = 16
NW = NC * NS
ROW = 256
RPW = 40
ROWS = NW * RPW
EP = ROWS * ROW
NPAD = 10240
RZ = NPAD // NS
GK = 4
NGRP = RPW // GK

_mesh = plsc.VectorSubcoreMesh(core_axis_name="c", subcore_axis_name="s")
_sc_params = pltpu.CompilerParams(use_tc_tiling_on_sc=False)


@functools.partial(
    pl.kernel,
    out_type=jax.ShapeDtypeStruct((NC, NPAD), jnp.float32),
    mesh=_mesh,
    compiler_params=_sc_params,
    scratch_types=[
        pltpu.VMEM((RPW, ROW), jnp.int32),
        pltpu.VMEM((ROW,), jnp.float32),
        pltpu.VMEM((RZ,), jnp.float32),
        pltpu.VMEM_SHARED((NPAD,), jnp.float32),
        pltpu.SemaphoreType.DMA,
    ],
)
def _sc_count(dst_hbm, out_hbm, dst_v, ones_v, zer_v, cnt_sh, csem):
    c = lax.axis_index("c")
    s = lax.axis_index("s")
    wid = s * NC + c
    zero16 = jnp.zeros((16,), jnp.float32)
    ones16 = jnp.ones((16,), jnp.float32)

    ld_dst = pltpu.async_copy(dst_hbm.at[pl.ds(wid * RPW, RPW)], dst_v, csem)

    def fill(i, _):
        zer_v[pl.ds(i * 16, 16)] = zero16
        return 0

    lax.fori_loop(0, RZ // 16, fill, 0)
    for k in range(ROW // 16):
        ones_v[pl.ds(k * 16, 16)] = ones16
    pltpu.sync_copy(zer_v, cnt_sh.at[pl.ds(s * RZ, RZ)])
    ld_dst.wait()
    plsc.subcore_barrier()

    def body(g, _):
        descs = [
            pltpu.async_copy(ones_v, cnt_sh.at[dst_v.at[g * 8 + t]],
                             csem, add=True)
            for t in range(8)
        ]
        for d in descs:
            d.wait()
        return 0

    lax.fori_loop(0, RPW // 8, body, 0)
    plsc.subcore_barrier()
    pltpu.sync_copy(cnt_sh.at[pl.ds(s * RZ, RZ)],
                    out_hbm.at[c].at[pl.ds(s * RZ, RZ)])


@functools.partial(
    pl.kernel,
    out_type=jax.ShapeDtypeStruct((NC, NPAD, D_HID), jnp.float32),
    mesh=_mesh,
    compiler_params=_sc_params,
    scratch_types=[
        pltpu.VMEM((RPW, ROW), jnp.int32),
        pltpu.VMEM((RPW, ROW), jnp.int32),
        pltpu.VMEM((2, GK, ROW, D_HID), jnp.float32),
        pltpu.VMEM_SHARED((NPAD, D_HID), jnp.float32),
        pltpu.VMEM_SHARED((NPAD, D_HID), jnp.float32),
        pltpu.SemaphoreType.DMA,
        pltpu.SemaphoreType.DMA,
        pltpu.SemaphoreType.DMA,
        pltpu.SemaphoreType.DMA,
    ],
)
def _sc_msgpass(hp_hbm, src_hbm, dst_hbm, out_hbm,
                src_v, dst_v, bufs, agg_sh, hp_sh,
                gsem0, gsem1, ssem0, ssem1):
    c = lax.axis_index("c")
    s = lax.axis_index("s")
    wid = s * NC + c
    zero16 = jnp.zeros((16,), jnp.float32)
    zbuf = bufs.at[0].at[0]

    def zbody(j, _):
        r = zbuf.at[j]
        r[pl.ds(0, 16)] = zero16
        r[pl.ds(16, 16)] = zero16
        return 0

    stage = pltpu.async_copy(hp_hbm.at[pl.ds(s * RZ, RZ)],
                             hp_sh.at[pl.ds(s * RZ, RZ)], gsem0)
    ld_src = pltpu.async_copy(src_hbm.at[pl.ds(wid * RPW, RPW)], src_v, ssem0)
    ld_dst = pltpu.async_copy(dst_hbm.at[pl.ds(wid * RPW, RPW)], dst_v, ssem1)
    lax.fori_loop(0, ROW, zbody, 0)
    for t in range(RZ // ROW):
        pltpu.sync_copy(zbuf, agg_sh.at[pl.ds(s * RZ + t * ROW, ROW)])
    if RZ % ROW:
        pltpu.sync_copy(zbuf.at[pl.ds(0, RZ % ROW)],
                        agg_sh.at[pl.ds(s * RZ + (RZ // ROW) * ROW, RZ % ROW)])
    stage.wait()
    ld_src.wait()
    ld_dst.wait()
    plsc.subcore_barrier()

    gsems = (gsem0, gsem1)

    def gather_group(g, bank):
        for t in range(GK):
            pltpu.async_copy(hp_sh.at[src_v.at[g * GK + t]],
                             bufs.at[bank].at[t], gsems[bank])

    def wait_gathers(g, bank):
        for t in range(GK):
            pltpu.make_async_copy(hp_sh.at[src_v.at[g * GK + t]],
                                  bufs.at[bank].at[t], gsems[bank]).wait()

    def scatter_group(g, bank, ssem):
        descs = [
            pltpu.async_copy(bufs.at[bank].at[t],
                             agg_sh.at[dst_v.at[g * GK + t]], ssem, add=True)
            for t in range(GK)
        ]
        for d in descs:
            d.wait()

    gather_group(0, 0)

    def body(i, _):
        g = 2 * i
        gather_group(g + 1, 1)
        wait_gathers(g, 0)
        scatter_group(g, 0, ssem0)
        gather_group(g + 2, 0)
        wait_gathers(g + 1, 1)
        scatter_group(g + 1, 1, ssem1)
        return 0

    lax.fori_loop(0, NGRP // 2 - 1, body, 0)
    gather_group(NGRP - 1, 1)
    wait_gathers(NGRP - 2, 0)
    scatter_group(NGRP - 2, 0, ssem0)
    wait_gathers(NGRP - 1, 1)
    scatter_group(NGRP - 1, 1, ssem1)

    plsc.subcore_barrier()
    pltpu.sync_copy(agg_sh.at[pl.ds(s * RZ, RZ)],
                    out_hbm.at[c].at[pl.ds(s * RZ, RZ)])


def _tc_prescale_body(x_ref, w1_ref, cnt_ref, hp_ref, dinv_ref):
    xw = jnp.dot(x_ref[...], w1_ref[...], preferred_element_type=jnp.float32)
    cnt = cnt_ref[0, :] + cnt_ref[1, :]
    dinv = lax.rsqrt(cnt[:N_NODES] + 1.0)[:, None]
    hp_ref[:N_NODES, :] = xw * dinv
    dinv_ref[...] = dinv


def _tc_layer2_body(agg_ref, hp1_ref, dinv_ref, b1_ref, w2_ref, hp2_ref):
    agg = agg_ref[0, :N_NODES, :] + agg_ref[1, :N_NODES, :]
    conv1 = dinv_ref[...] * (agg + hp1_ref[:N_NODES, :]) + b1_ref[...]
    h1 = jnp.maximum(conv1, 0.0)
    xw2 = jnp.dot(h1, w2_ref[...], preferred_element_type=jnp.float32)
    hp2_ref[:N_NODES, :] = xw2 * dinv_ref[...]


def _tc_head_body(agg_ref, hp2_ref, dinv_ref, b2_ref, batch_ref,
                  wc_ref, bc_ref, out_ref):
    agg = agg_ref[0, :N_NODES, :] + agg_ref[1, :N_NODES, :]
    conv2 = dinv_ref[...] * (agg + hp2_ref[:N_NODES, :]) + b2_ref[...]
    gids = lax.broadcasted_iota(jnp.int32, (N_GRAPHS, N_NODES), 0)
    mask = (batch_ref[...] == gids).astype(jnp.float32)
    sums = jnp.dot(mask, conv2, preferred_element_type=jnp.float32)
    cnt = jnp.sum(mask, axis=1, keepdims=True)
    pooled = sums / jnp.maximum(cnt, 1.0)
    out_ref[...] = (
        jnp.dot(pooled, wc_ref[...], preferred_element_type=jnp.float32)
        + bc_ref[...])


@jax.jit
def kernel(x, edge_index, batch, W1, b1, W2, b2, Wc, bc):
    src = edge_index[0].astype(jnp.int32)
    dst = edge_index[1].astype(jnp.int32)
    pad = EP - N_EDGES
    src2d = jnp.concatenate(
        [src, jnp.zeros((pad,), jnp.int32)]).reshape(ROWS, ROW)
    dst2d = jnp.concatenate(
        [dst, jnp.full((pad,), N_NODES, jnp.int32)]).reshape(ROWS, ROW)

    cnt_parts = _sc_count(dst2d)

    hp1, dinv = pl.pallas_call(
        _tc_prescale_body,
        out_shape=(
            jax.ShapeDtypeStruct((NPAD, D_HID), jnp.float32),
            jax.ShapeDtypeStruct((N_NODES, 1), jnp.float32),
        ),
    )(x, W1, cnt_parts)

    agg1 = _sc_msgpass(hp1, src2d, dst2d)

    hp2 = pl.pallas_call(
        _tc_layer2_body,
        out_shape=jax.ShapeDtypeStruct((NPAD, D_HID), jnp.float32),
    )(agg1, hp1, dinv, b1.reshape(1, D_HID), W2)

    agg2 = _sc_msgpass(hp2, src2d, dst2d)

    out = pl.pallas_call(
        _tc_head_body,
        out_shape=jax.ShapeDtypeStruct((N_GRAPHS, bc.shape[0]), jnp.float32),
    )(agg2, hp2, dinv, b2.reshape(1, D_HID), batch.astype(jnp.int32)[None, :],
      Wc, bc.reshape(1, bc.shape[0]))
    return out
